# baseline (speedup 1.0000x reference)
import functools

import jax
import jax.numpy as jnp
from jax.experimental import pallas as pl
from jax.experimental.pallas import tpu as pltpu


def _moe_body(x_ref, wgt_ref, w1_ref, b1_ref, w2_ref, b2_ref, out_ref,
              xb_ref, i1_ref, i2_ref, a1_ref, a2_ref, *, num_experts):
    e = pl.program_id(0)

    @pl.when(e == 0)
    def _router():
        x = x_ref[...]
        xb_ref[...] = x.astype(jnp.bfloat16)
        logits = jnp.dot(x, wgt_ref[...], preferred_element_type=jnp.float32)
        lane = jax.lax.broadcasted_iota(jnp.int32, logits.shape, 1)
        valid = lane < num_experts
        logits = jnp.where(valid, logits, jnp.float32(-1e30))
        mx = jnp.max(logits, axis=1, keepdims=True)
        ex = jnp.where(valid, jnp.exp(logits - mx), 0.0)
        probs = ex / jnp.sum(ex, axis=1, keepdims=True)
        big = jnp.int32(logits.shape[1])
        p1 = jnp.max(probs, axis=1, keepdims=True)
        i1 = jnp.min(jnp.where(probs == p1, lane, big), axis=1, keepdims=True)
        probs2 = jnp.where(lane == i1, jnp.float32(-1.0), probs)
        p2 = jnp.max(probs2, axis=1, keepdims=True)
        i2 = jnp.min(jnp.where(probs2 == p2, lane, big), axis=1, keepdims=True)
        s = p1 + p2
        i1_ref[...] = i1
        i2_ref[...] = i2
        a1_ref[...] = p1 / s
        a2_ref[...] = p2 / s

    c = (jnp.where(i1_ref[...] == e, a1_ref[...], 0.0)
         + jnp.where(i2_ref[...] == e, a2_ref[...], 0.0))

    w1b = w1_ref[0].astype(jnp.bfloat16)
    w2b = w2_ref[0].astype(jnp.bfloat16)
    b1v = b1_ref[0]
    b2v = b2_ref[0]
    TC_CHUNK = 512
    n_tok = xb_ref.shape[0]
    for t0 in range(0, n_tok, TC_CHUNK):
        sl = pl.ds(t0, TC_CHUNK)
        h = jnp.dot(xb_ref[sl, :], w1b, preferred_element_type=jnp.float32)
        h = h + b1v
        h = 0.5 * h * (1.0 + jax.lax.erf(h * jnp.float32(0.7071067811865476)))
        y = jnp.dot(h.astype(jnp.bfloat16), w2b,
                    preferred_element_type=jnp.float32)
        v = c[t0:t0 + TC_CHUNK] * (y + b2v)

        @pl.when(e == 0)
        def _init():
            out_ref[sl, :] = v

        @pl.when(e != 0)
        def _accum():
            out_ref[sl, :] += v


def kernel(hidden_states, Wg, W1, b1, W2, b2):
    B, S, D = hidden_states.shape
    E, _, F = W1.shape
    T = B * S
    x = hidden_states.reshape(T, D)

    wgt = jnp.zeros((D, 128), jnp.float32).at[:, :E].set(Wg.T)

    body = functools.partial(_moe_body, num_experts=E)
    out = pl.pallas_call(
        body,
        grid=(E,),
        in_specs=[
            pl.BlockSpec((T, D), lambda e: (0, 0)),
            pl.BlockSpec((D, 128), lambda e: (0, 0)),
            pl.BlockSpec((1, D, F), lambda e: (e, 0, 0)),
            pl.BlockSpec((1, 1, F), lambda e: (e, 0, 0)),
            pl.BlockSpec((1, F, D), lambda e: (e, 0, 0)),
            pl.BlockSpec((1, 1, D), lambda e: (e, 0, 0)),
        ],
        out_specs=pl.BlockSpec((T, D), lambda e: (0, 0)),
        out_shape=jax.ShapeDtypeStruct((T, D), jnp.float32),
        scratch_shapes=[
            pltpu.VMEM((T, D), jnp.bfloat16),
            pltpu.VMEM((T, 1), jnp.int32),
            pltpu.VMEM((T, 1), jnp.int32),
            pltpu.VMEM((T, 1), jnp.float32),
            pltpu.VMEM((T, 1), jnp.float32),
        ],
        compiler_params=pltpu.CompilerParams(
            dimension_semantics=("arbitrary",),
        ),
    )(x, wgt, W1, b1.reshape(E, 1, F), W2, b2.reshape(E, 1, D))
    return out.reshape(B, S, D)


# speedup vs baseline: 1.0101x; 1.0101x over previous
import functools

import jax
import jax.numpy as jnp
from jax.experimental import pallas as pl
from jax.experimental.pallas import tpu as pltpu


def _moe_body(x_ref, wgt_ref, w1_ref, b1_ref, w2_ref, b2_ref, out_ref,
              xb_ref, i1_ref, i2_ref, a1_ref, a2_ref, *, num_experts):
    e = pl.program_id(1)

    @pl.when(e == 0)
    def _router():
        x = x_ref[...]
        xb_ref[...] = x.astype(jnp.bfloat16)
        logits = jnp.dot(x, wgt_ref[...], preferred_element_type=jnp.float32)
        lane = jax.lax.broadcasted_iota(jnp.int32, logits.shape, 1)
        valid = lane < num_experts
        logits = jnp.where(valid, logits, jnp.float32(-1e30))
        mx = jnp.max(logits, axis=1, keepdims=True)
        ex = jnp.where(valid, jnp.exp(logits - mx), 0.0)
        probs = ex / jnp.sum(ex, axis=1, keepdims=True)
        big = jnp.int32(logits.shape[1])
        p1 = jnp.max(probs, axis=1, keepdims=True)
        i1 = jnp.min(jnp.where(probs == p1, lane, big), axis=1, keepdims=True)
        probs2 = jnp.where(lane == i1, jnp.float32(-1.0), probs)
        p2 = jnp.max(probs2, axis=1, keepdims=True)
        i2 = jnp.min(jnp.where(probs2 == p2, lane, big), axis=1, keepdims=True)
        s = p1 + p2
        i1_ref[...] = i1
        i2_ref[...] = i2
        a1_ref[...] = p1 / s
        a2_ref[...] = p2 / s

    c = (jnp.where(i1_ref[...] == e, a1_ref[...], 0.0)
         + jnp.where(i2_ref[...] == e, a2_ref[...], 0.0))

    w1b = w1_ref[0].astype(jnp.bfloat16)
    w2b = w2_ref[0].astype(jnp.bfloat16)
    b1v = b1_ref[0]
    b2v = b2_ref[0]
    TC_CHUNK = 512
    n_tok = xb_ref.shape[0]
    for t0 in range(0, n_tok, TC_CHUNK):
        sl = pl.ds(t0, TC_CHUNK)
        h = jnp.dot(xb_ref[sl, :], w1b, preferred_element_type=jnp.float32)
        h = h + b1v
        h = 0.5 * h * (1.0 + jax.lax.erf(h * jnp.float32(0.7071067811865476)))
        y = jnp.dot(h.astype(jnp.bfloat16), w2b,
                    preferred_element_type=jnp.float32)
        v = c[t0:t0 + TC_CHUNK] * (y + b2v)

        @pl.when(e == 0)
        def _init():
            out_ref[sl, :] = v

        @pl.when(e != 0)
        def _accum():
            out_ref[sl, :] += v


def kernel(hidden_states, Wg, W1, b1, W2, b2):
    B, S, D = hidden_states.shape
    E, _, F = W1.shape
    T = B * S
    x = hidden_states.reshape(T, D)

    TB = T // 2

    wgt = jnp.zeros((D, 128), jnp.float32).at[:, :E].set(Wg.T)

    body = functools.partial(_moe_body, num_experts=E)
    out = pl.pallas_call(
        body,
        grid=(T // TB, E),
        in_specs=[
            pl.BlockSpec((TB, D), lambda t, e: (t, 0)),
            pl.BlockSpec((D, 128), lambda t, e: (0, 0)),
            pl.BlockSpec((1, D, F), lambda t, e: (e, 0, 0)),
            pl.BlockSpec((1, 1, F), lambda t, e: (e, 0, 0)),
            pl.BlockSpec((1, F, D), lambda t, e: (e, 0, 0)),
            pl.BlockSpec((1, 1, D), lambda t, e: (e, 0, 0)),
        ],
        out_specs=pl.BlockSpec((TB, D), lambda t, e: (t, 0)),
        out_shape=jax.ShapeDtypeStruct((T, D), jnp.float32),
        scratch_shapes=[
            pltpu.VMEM((TB, D), jnp.bfloat16),
            pltpu.VMEM((TB, 1), jnp.int32),
            pltpu.VMEM((TB, 1), jnp.int32),
            pltpu.VMEM((TB, 1), jnp.float32),
            pltpu.VMEM((TB, 1), jnp.float32),
        ],
        compiler_params=pltpu.CompilerParams(
            dimension_semantics=("parallel", "arbitrary"),
        ),
    )(x, wgt, W1, b1.reshape(E, 1, F), W2, b2.reshape(E, 1, D))
    return out.reshape(B, S, D)


# speedup vs baseline: 1.0782x; 1.0675x over previous
import functools

import jax
import jax.numpy as jnp
from jax.experimental import pallas as pl
from jax.experimental.pallas import tpu as pltpu


def _moe_body(x_ref, wgt_ref, w1_ref, b1_ref, w2_ref, b2_ref, out_ref,
              xb_ref, i1_ref, i2_ref, a1_ref, a2_ref, *, num_experts):
    e = pl.program_id(1)

    @pl.when(e == 0)
    def _router():
        x = x_ref[...]
        xb_ref[...] = x.astype(jnp.bfloat16)
        logits = jnp.dot(x, wgt_ref[...], preferred_element_type=jnp.float32)
        lane = jax.lax.broadcasted_iota(jnp.int32, logits.shape, 1)
        valid = lane < num_experts
        logits = jnp.where(valid, logits, jnp.float32(-1e30))
        mx = jnp.max(logits, axis=1, keepdims=True)
        ex = jnp.where(valid, jnp.exp(logits - mx), 0.0)
        probs = ex / jnp.sum(ex, axis=1, keepdims=True)
        big = jnp.int32(logits.shape[1])
        p1 = jnp.max(probs, axis=1, keepdims=True)
        i1 = jnp.min(jnp.where(probs == p1, lane, big), axis=1, keepdims=True)
        probs2 = jnp.where(lane == i1, jnp.float32(-1.0), probs)
        p2 = jnp.max(probs2, axis=1, keepdims=True)
        i2 = jnp.min(jnp.where(probs2 == p2, lane, big), axis=1, keepdims=True)
        s = p1 + p2
        i1_ref[...] = i1
        i2_ref[...] = i2
        a1_ref[...] = p1 / s
        a2_ref[...] = p2 / s

    c = (jnp.where(i1_ref[...] == e, a1_ref[...], 0.0)
         + jnp.where(i2_ref[...] == e, a2_ref[...], 0.0))

    w1b = w1_ref[0].astype(jnp.bfloat16)
    w2b = w2_ref[0].astype(jnp.bfloat16)
    b1v = b1_ref[0]
    b2v = b2_ref[0]
    h = jnp.dot(xb_ref[...], w1b, preferred_element_type=jnp.float32)
    h = h + b1v
    h = 0.5 * h * (1.0 + jax.lax.erf(h * jnp.float32(0.7071067811865476)))
    y = jnp.dot(h.astype(jnp.bfloat16), w2b,
                preferred_element_type=jnp.float32)
    v = c * (y + b2v)

    @pl.when(e == 0)
    def _init():
        out_ref[...] = v

    @pl.when(e != 0)
    def _accum():
        out_ref[...] += v


def kernel(hidden_states, Wg, W1, b1, W2, b2):
    B, S, D = hidden_states.shape
    E, _, F = W1.shape
    T = B * S
    x = hidden_states.reshape(T, D)

    TB = T // 2

    wgt = jnp.zeros((D, 128), jnp.float32).at[:, :E].set(Wg.T)

    body = functools.partial(_moe_body, num_experts=E)
    out = pl.pallas_call(
        body,
        grid=(T // TB, E),
        in_specs=[
            pl.BlockSpec((TB, D), lambda t, e: (t, 0)),
            pl.BlockSpec((D, 128), lambda t, e: (0, 0)),
            pl.BlockSpec((1, D, F), lambda t, e: (e, 0, 0)),
            pl.BlockSpec((1, 1, F), lambda t, e: (e, 0, 0)),
            pl.BlockSpec((1, F, D), lambda t, e: (e, 0, 0)),
            pl.BlockSpec((1, 1, D), lambda t, e: (e, 0, 0)),
        ],
        out_specs=pl.BlockSpec((TB, D), lambda t, e: (t, 0)),
        out_shape=jax.ShapeDtypeStruct((T, D), jnp.float32),
        scratch_shapes=[
            pltpu.VMEM((TB, D), jnp.bfloat16),
            pltpu.VMEM((TB, 1), jnp.int32),
            pltpu.VMEM((TB, 1), jnp.int32),
            pltpu.VMEM((TB, 1), jnp.float32),
            pltpu.VMEM((TB, 1), jnp.float32),
        ],
        compiler_params=pltpu.CompilerParams(
            dimension_semantics=("parallel", "arbitrary"),
        ),
    )(x, wgt, W1, b1.reshape(E, 1, F), W2, b2.reshape(E, 1, D))
    return out.reshape(B, S, D)
